# baseline (speedup 1.0000x reference)
import jax
import jax.numpy as jnp
from jax.experimental import pallas as pl
from jax.experimental.pallas import tpu as pltpu

K = 8192
D = 256
_NCH = 16
_CRW = K // _NCH


def _body(x_hbm, o_hbm, buf, in_sems, out_sems):
    for c in range(_NCH):
        pltpu.make_async_copy(
            x_hbm.at[pl.ds(c * _CRW, _CRW)], buf.at[c], in_sems.at[c]).start()
    for c in range(_NCH):
        pltpu.make_async_copy(
            x_hbm.at[pl.ds(c * _CRW, _CRW)], buf.at[c], in_sems.at[c]).wait()
        x = buf[c]
        sq = x * x
        ones = jnp.ones((D, 1), jnp.float32)
        s = jax.lax.dot_general(sq, ones, (((1,), (0,)), ((), ())),
                                preferred_element_type=jnp.float32)
        inv = 1.0 / jnp.maximum(jnp.sqrt(s), 1e-12)
        buf[c] = x * inv
        pltpu.make_async_copy(
            buf.at[c], o_hbm.at[pl.ds(c * _CRW, _CRW)], out_sems.at[c]).start()
    for c in range(_NCH):
        pltpu.make_async_copy(
            buf.at[c], o_hbm.at[pl.ds(c * _CRW, _CRW)], out_sems.at[c]).wait()


def kernel(prototypes):
    return pl.pallas_call(
        _body,
        in_specs=[pl.BlockSpec(memory_space=pl.ANY)],
        out_specs=pl.BlockSpec(memory_space=pl.ANY),
        out_shape=jax.ShapeDtypeStruct((K, D), jnp.float32),
        scratch_shapes=[
            pltpu.VMEM((_NCH, _CRW, D), jnp.float32),
            pltpu.SemaphoreType.DMA((_NCH,)),
            pltpu.SemaphoreType.DMA((_NCH,)),
        ],
    )(prototypes)
